# baseline (speedup 1.0000x reference)
import functools

import jax
import jax.numpy as jnp
from jax import lax
from jax.experimental import pallas as pl
from jax.experimental.pallas import tpu as pltpu
from jax.experimental.pallas import tpu_sc as plsc

N = 50000
E = 800000
D = 100
LAYERS = 3

L = 16
DP = 128
NSL = 7
RB = 528
NB = 96
NP = NB * RB
NC = 2
NS = 16
NW = NC * NS
BPW = NB // NW
CH = 128
BBL = 112


def _sc_layer_body(x_hbm, cs_hbm, vs_hbm, rs_hbm, bb_hbm, out_hbm,
                   bb_v, idx_a, idx_b, val_a, val_b, row_a, row_b,
                   st_a, st_b, acc, gsem_a, gsem_b):
    wid = lax.axis_index("s") * NC + lax.axis_index("c")
    idx_m = [idx_a, idx_b]
    val_m = [val_a, val_b]
    row_m = [row_a, row_b]
    st_m = [st_a, st_b]
    gsem = [gsem_a, gsem_b]

    pltpu.sync_copy(bb_hbm, bb_v)

    def read_bb(i):
        v = bb_v[pl.ds(i, L)]
        return v[0]

    zeros = jnp.zeros((L,), jnp.float32)

    for k in range(BPW):
        b = wid * BPW + k
        row_base = b * RB
        e_start = read_bb(b)
        e_end = read_bb(b + 1)
        ca = (e_start // CH) * CH
        nch = (e_end - ca + CH - 1) // CH

        @plsc.parallel_loop(0, RB, unroll=4)
        def _(r):
            for j in range(NSL):
                acc[r, pl.ds(j * L, L)] = zeros

        def load_meta(c, m):
            base = ca + c * CH
            pltpu.sync_copy(cs_hbm.at[pl.ds(base, CH)], idx_m[m])
            pltpu.sync_copy(vs_hbm.at[pl.ds(base, CH)], val_m[m].at[pl.ds(0, CH)])
            pltpu.sync_copy(rs_hbm.at[pl.ds(base, CH)], row_m[m].at[pl.ds(0, CH)])

        def gather(m):
            return pltpu.make_async_copy(x_hbm.at[idx_m[m]], st_m[m], gsem[m])

        @pl.when(nch > 0)
        def _():
            load_meta(0, 0)
            gather(0).start()

        def cpair(c2, _):
            for p in range(2):
                c = c2 * 2 + p

                @pl.when(c < nch)
                def _(c=c, p=p):
                    @pl.when(c + 1 < nch)
                    def _():
                        load_meta(c + 1, 1 - p)
                        gather(1 - p).start()

                    gather(p).wait()

                    base = ca + c * CH
                    lo = jnp.maximum(e_start - base, 0)
                    hi = jnp.minimum(e_end - base, CH)
                    stage = st_m[p]
                    valr = val_m[p]
                    rowr = row_m[p]

                    @plsc.parallel_loop(lo, hi, unroll=6)
                    def _(e):
                        rv = rowr[pl.ds(e, L)]
                        r0 = rv[0] - row_base
                        vv = valr[pl.ds(e, L)]
                        v0 = vv[0]
                        for t in range(NSL):
                            gv = stage[e, pl.ds(t * L, L)]
                            plsc.addupdate(acc.at[r0, pl.ds(t * L, L)], gv * v0)
            return 0

        lax.fori_loop(0, (nch + 1) // 2, cpair, 0)

        pltpu.sync_copy(acc, out_hbm.at[pl.ds(row_base, RB)])


@jax.jit
def _sc_layer(x, cs, vs, rs, bb):
    mesh = plsc.VectorSubcoreMesh(core_axis_name="c", subcore_axis_name="s")
    return pl.kernel(
        _sc_layer_body,
        out_type=jax.ShapeDtypeStruct((NP, DP), jnp.float32),
        mesh=mesh,
        scratch_types=[
            pltpu.VMEM((BBL,), jnp.int32),
            pltpu.VMEM((CH,), jnp.int32),
            pltpu.VMEM((CH,), jnp.int32),
            pltpu.VMEM((CH + L,), jnp.float32),
            pltpu.VMEM((CH + L,), jnp.float32),
            pltpu.VMEM((CH + L,), jnp.int32),
            pltpu.VMEM((CH + L,), jnp.int32),
            pltpu.VMEM((CH, DP), jnp.float32),
            pltpu.VMEM((CH, DP), jnp.float32),
            pltpu.VMEM((RB, DP), jnp.float32),
            pltpu.SemaphoreType.DMA,
            pltpu.SemaphoreType.DMA,
        ],
    )(x, cs, vs, rs, bb)


def _add4_body(a, b, c, d, o):
    o[...] = (a[...] + b[...]) + (c[...] + d[...])


@jax.jit
def _add4(a, b, c, d):
    blk = 384
    return pl.pallas_call(
        _add4_body,
        out_shape=jax.ShapeDtypeStruct((NP, DP), jnp.float32),
        grid=(NP // blk,),
        in_specs=[pl.BlockSpec((blk, DP), lambda i: (i, 0))] * 4,
        out_specs=pl.BlockSpec((blk, DP), lambda i: (i, 0)),
    )(a, b, c, d)


def kernel(adj_indices, adj_values, embedding):
    row = adj_indices[0]
    col = adj_indices[1]
    rs, cs, vs = lax.sort((row, col, adj_values), is_stable=False, num_keys=1)
    bounds = (jnp.arange(NB + 1, dtype=jnp.int32) * RB)
    bb = jnp.searchsorted(rs, bounds).astype(jnp.int32)
    bb = jnp.concatenate([bb, jnp.full((BBL - NB - 1,), E, jnp.int32)])

    x0 = jnp.pad(embedding, ((0, NP - N), (0, DP - D)))
    x1 = _sc_layer(x0, cs, vs, rs, bb)
    x2 = _sc_layer(x1, cs, vs, rs, bb)
    x3 = _sc_layer(x2, cs, vs, rs, bb)
    final = _add4(x0, x1, x2, x3)
    return final[:N, :D]


# speedup vs baseline: 1.1870x; 1.1870x over previous
import functools

import jax
import jax.numpy as jnp
from jax import lax
from jax.experimental import pallas as pl
from jax.experimental.pallas import tpu as pltpu
from jax.experimental.pallas import tpu_sc as plsc

N = 50000
E = 800000
D = 100
LAYERS = 3

L = 16
DP = 128
NSL = 7
RB = 528
NB = 96
NP = NB * RB
NC = 2
NS = 16
NW = NC * NS
BPW = NB // NW
CH = 128
BBL = 112


def _sc_layer_body(x_hbm, ks_hbm, vs_hbm, bb_hbm, out_hbm,
                   bb_v, idx_a, idx_b, val_a, val_b, key_a, key_b,
                   st_a, st_b, acc, gsem_a, gsem_b):
    wid = lax.axis_index("s") * NC + lax.axis_index("c")
    idx_m = [idx_a, idx_b]
    val_m = [val_a, val_b]
    key_m = [key_a, key_b]
    st_m = [st_a, st_b]
    gsem = [gsem_a, gsem_b]
    lomask = jnp.full((L,), 0xFFFF, jnp.uint32)

    pltpu.sync_copy(bb_hbm, bb_v)

    def read_bb(i):
        v = bb_v[pl.ds(i, L)]
        return v[0]

    zeros = jnp.zeros((L,), jnp.float32)

    for k in range(BPW):
        b = wid * BPW + k
        row_base = b * RB
        e_start = read_bb(b)
        e_end = read_bb(b + 1)
        ca = (e_start // CH) * CH
        nch = (e_end - ca + CH - 1) // CH

        @plsc.parallel_loop(0, RB, unroll=4)
        def _(r):
            for j in range(NSL):
                acc[r, pl.ds(j * L, L)] = zeros

        def load_meta(c, m):
            base = ca + c * CH
            pltpu.sync_copy(ks_hbm.at[pl.ds(base, CH)], key_m[m].at[pl.ds(0, CH)])
            pltpu.sync_copy(vs_hbm.at[pl.ds(base, CH)], val_m[m].at[pl.ds(0, CH)])
            for q in range(CH // L):
                kvec = key_m[m][pl.ds(q * L, L)]
                idx_m[m][pl.ds(q * L, L)] = lax.convert_element_type(
                    kvec & lomask, jnp.int32)

        def gather(m):
            return pltpu.make_async_copy(x_hbm.at[idx_m[m]], st_m[m], gsem[m])

        @pl.when(nch > 0)
        def _():
            load_meta(0, 0)
            gather(0).start()

        def cpair(c2, _):
            for p in range(2):
                c = c2 * 2 + p

                @pl.when(c < nch)
                def _(c=c, p=p):
                    @pl.when(c + 1 < nch)
                    def _():
                        load_meta(c + 1, 1 - p)
                        gather(1 - p).start()

                    gather(p).wait()

                    base = ca + c * CH
                    lo = jnp.maximum(e_start - base, 0)
                    hi = jnp.minimum(e_end - base, CH)
                    stage = st_m[p]
                    valr = val_m[p]
                    keyr = key_m[p]
                    row_base_u = lax.convert_element_type(row_base, jnp.uint32)

                    @plsc.parallel_loop(lo, hi, unroll=4)
                    def _(e):
                        kv = keyr[pl.ds(e, L)]
                        r0 = lax.convert_element_type(
                            (kv[0] >> 16) - row_base_u, jnp.int32)
                        vv = valr[pl.ds(e, L)]
                        v0 = vv[0]
                        for t in range(NSL):
                            gv = stage[e, pl.ds(t * L, L)]
                            plsc.addupdate(acc.at[r0, pl.ds(t * L, L)], gv * v0)
            return 0

        lax.fori_loop(0, (nch + 1) // 2, cpair, 0)

        pltpu.sync_copy(acc, out_hbm.at[pl.ds(row_base, RB)])


@jax.jit
def _sc_layer(x, ks, vs, bb):
    mesh = plsc.VectorSubcoreMesh(core_axis_name="c", subcore_axis_name="s")
    return pl.kernel(
        _sc_layer_body,
        out_type=jax.ShapeDtypeStruct((NP, DP), jnp.float32),
        mesh=mesh,
        scratch_types=[
            pltpu.VMEM((BBL,), jnp.int32),
            pltpu.VMEM((CH,), jnp.int32),
            pltpu.VMEM((CH,), jnp.int32),
            pltpu.VMEM((CH + L,), jnp.float32),
            pltpu.VMEM((CH + L,), jnp.float32),
            pltpu.VMEM((CH + L,), jnp.uint32),
            pltpu.VMEM((CH + L,), jnp.uint32),
            pltpu.VMEM((CH, DP), jnp.float32),
            pltpu.VMEM((CH, DP), jnp.float32),
            pltpu.VMEM((RB, DP), jnp.float32),
            pltpu.SemaphoreType.DMA,
            pltpu.SemaphoreType.DMA,
        ],
    )(x, ks, vs, bb)


def _add4_body(a, b, c, d, o):
    o[...] = (a[...] + b[...]) + (c[...] + d[...])


@jax.jit
def _add4(a, b, c, d):
    blk = 384
    return pl.pallas_call(
        _add4_body,
        out_shape=jax.ShapeDtypeStruct((NP, DP), jnp.float32),
        grid=(NP // blk,),
        in_specs=[pl.BlockSpec((blk, DP), lambda i: (i, 0))] * 4,
        out_specs=pl.BlockSpec((blk, DP), lambda i: (i, 0)),
    )(a, b, c, d)


def kernel(adj_indices, adj_values, embedding):
    row = adj_indices[0]
    col = adj_indices[1]
    key = (row.astype(jnp.uint32) << 16) | col.astype(jnp.uint32)
    ks, vs = lax.sort((key, adj_values), is_stable=False, num_keys=1)
    bounds = (jnp.arange(NB + 1, dtype=jnp.uint32) * RB) << 16
    bb = jnp.searchsorted(ks, bounds).astype(jnp.int32)
    bb = jnp.concatenate([bb, jnp.full((BBL - NB - 1,), E, jnp.int32)])

    x0 = jnp.pad(embedding, ((0, NP - N), (0, DP - D)))
    x1 = _sc_layer(x0, ks, vs, bb)
    x2 = _sc_layer(x1, ks, vs, bb)
    x3 = _sc_layer(x2, ks, vs, bb)
    final = _add4(x0, x1, x2, x3)
    return final[:N, :D]


# speedup vs baseline: 1.1913x; 1.0036x over previous
import functools

import jax
import jax.numpy as jnp
from jax import lax
from jax.experimental import pallas as pl
from jax.experimental.pallas import tpu as pltpu
from jax.experimental.pallas import tpu_sc as plsc

N = 50000
E = 800000
D = 100
LAYERS = 3

L = 16
DP = 128
NSL = 7
RB = 528
NB = 96
NP = NB * RB
NC = 2
NS = 16
NW = NC * NS
BPW = NB // NW
CH = 128
BBL = 112


def _sc_layer_body(x_hbm, ks_hbm, vs_hbm, bb_hbm, out_hbm,
                   bb_v, idx_a, idx_b, idx_c, val_a, val_b, val_c,
                   key_a, key_b, key_c, st_a, st_b, st_c, acc,
                   gsem_a, gsem_b, gsem_c):
    wid = lax.axis_index("s") * NC + lax.axis_index("c")
    idx_m = [idx_a, idx_b, idx_c]
    val_m = [val_a, val_b, val_c]
    key_m = [key_a, key_b, key_c]
    st_m = [st_a, st_b, st_c]
    gsem = [gsem_a, gsem_b, gsem_c]
    lomask = jnp.full((L,), 0xFFFF, jnp.uint32)

    pltpu.sync_copy(bb_hbm, bb_v)

    def read_bb(i):
        v = bb_v[pl.ds(i, L)]
        return v[0]

    zeros = jnp.zeros((L,), jnp.float32)

    for k in range(BPW):
        b = wid * BPW + k
        row_base = b * RB
        e_start = read_bb(b)
        e_end = read_bb(b + 1)
        ca = (e_start // CH) * CH
        nch = (e_end - ca + CH - 1) // CH

        @plsc.parallel_loop(0, RB, unroll=4)
        def _(r):
            for j in range(NSL):
                acc[r, pl.ds(j * L, L)] = zeros

        def load_meta(c, m):
            base = ca + c * CH
            pltpu.sync_copy(ks_hbm.at[pl.ds(base, CH)], key_m[m].at[pl.ds(0, CH)])
            pltpu.sync_copy(vs_hbm.at[pl.ds(base, CH)], val_m[m].at[pl.ds(0, CH)])
            for q in range(CH // L):
                kvec = key_m[m][pl.ds(q * L, L)]
                idx_m[m][pl.ds(q * L, L)] = lax.convert_element_type(
                    kvec & lomask, jnp.int32)

        def gather(m):
            return pltpu.make_async_copy(x_hbm.at[idx_m[m]], st_m[m], gsem[m])

        @pl.when(nch > 0)
        def _():
            load_meta(0, 0)
            gather(0).start()

        @pl.when(nch > 1)
        def _():
            load_meta(1, 1)
            gather(1).start()

        def ctrip(c3, _):
            for p in range(3):
                c = c3 * 3 + p

                @pl.when(c < nch)
                def _(c=c, p=p):
                    pn = (p + 2) % 3

                    @pl.when(c + 2 < nch)
                    def _():
                        load_meta(c + 2, pn)
                        gather(pn).start()

                    gather(p).wait()

                    base = ca + c * CH
                    lo = jnp.maximum(e_start - base, 0)
                    hi = jnp.minimum(e_end - base, CH)
                    stage = st_m[p]
                    valr = val_m[p]
                    keyr = key_m[p]
                    row_base_u = lax.convert_element_type(row_base, jnp.uint32)

                    @plsc.parallel_loop(lo, hi, unroll=4)
                    def _(e):
                        kv = keyr[pl.ds(e, L)]
                        r0 = lax.convert_element_type(
                            (kv[0] >> 16) - row_base_u, jnp.int32)
                        vv = valr[pl.ds(e, L)]
                        v0 = vv[0]
                        for t in range(NSL):
                            gv = stage[e, pl.ds(t * L, L)]
                            plsc.addupdate(acc.at[r0, pl.ds(t * L, L)], gv * v0)
            return 0

        lax.fori_loop(0, (nch + 2) // 3, ctrip, 0)

        pltpu.sync_copy(acc, out_hbm.at[pl.ds(row_base, RB)])


@jax.jit
def _sc_layer(x, ks, vs, bb):
    mesh = plsc.VectorSubcoreMesh(core_axis_name="c", subcore_axis_name="s")
    return pl.kernel(
        _sc_layer_body,
        out_type=jax.ShapeDtypeStruct((NP, DP), jnp.float32),
        mesh=mesh,
        scratch_types=[
            pltpu.VMEM((BBL,), jnp.int32),
            pltpu.VMEM((CH,), jnp.int32),
            pltpu.VMEM((CH,), jnp.int32),
            pltpu.VMEM((CH,), jnp.int32),
            pltpu.VMEM((CH + L,), jnp.float32),
            pltpu.VMEM((CH + L,), jnp.float32),
            pltpu.VMEM((CH + L,), jnp.float32),
            pltpu.VMEM((CH + L,), jnp.uint32),
            pltpu.VMEM((CH + L,), jnp.uint32),
            pltpu.VMEM((CH + L,), jnp.uint32),
            pltpu.VMEM((CH, DP), jnp.float32),
            pltpu.VMEM((CH, DP), jnp.float32),
            pltpu.VMEM((CH, DP), jnp.float32),
            pltpu.VMEM((RB, DP), jnp.float32),
            pltpu.SemaphoreType.DMA,
            pltpu.SemaphoreType.DMA,
            pltpu.SemaphoreType.DMA,
        ],
    )(x, ks, vs, bb)


def _add4_body(a, b, c, d, o):
    o[...] = (a[...] + b[...]) + (c[...] + d[...])


@jax.jit
def _add4(a, b, c, d):
    blk = 384
    return pl.pallas_call(
        _add4_body,
        out_shape=jax.ShapeDtypeStruct((NP, DP), jnp.float32),
        grid=(NP // blk,),
        in_specs=[pl.BlockSpec((blk, DP), lambda i: (i, 0))] * 4,
        out_specs=pl.BlockSpec((blk, DP), lambda i: (i, 0)),
    )(a, b, c, d)


def kernel(adj_indices, adj_values, embedding):
    row = adj_indices[0]
    col = adj_indices[1]
    key = (row.astype(jnp.uint32) << 16) | col.astype(jnp.uint32)
    ks, vs = lax.sort((key, adj_values), is_stable=False, num_keys=1)
    bounds = (jnp.arange(NB + 1, dtype=jnp.uint32) * RB) << 16
    bb = jnp.searchsorted(ks, bounds).astype(jnp.int32)
    bb = jnp.concatenate([bb, jnp.full((BBL - NB - 1,), E, jnp.int32)])

    x0 = jnp.pad(embedding, ((0, NP - N), (0, DP - D)))
    x1 = _sc_layer(x0, ks, vs, bb)
    x2 = _sc_layer(x1, ks, vs, bb)
    x3 = _sc_layer(x2, ks, vs, bb)
    final = _add4(x0, x1, x2, x3)
    return final[:N, :D]
